# Initial kernel scaffold; baseline (speedup 1.0000x reference)
import jax
import jax.numpy as jnp
from jax.experimental import pallas as pl


def kernel(x, edge_index, batch_idx, emb, W1, a_src1, a_dst1, b1, ln_g, ln_b, W2, a_src2, a_dst2, b2, W3, b3):
    raise NotImplementedError("write your pallas kernel here")


# speedup vs baseline: 4.7088x; 4.7088x over previous
import functools

import jax
import jax.numpy as jnp
from jax import lax
from jax.experimental import pallas as pl
from jax.experimental.pallas import tpu as pltpu
from jax.experimental.pallas import tpu_sc as plsc

_N = 10000
_E = 320000
_HID = 128
_HEADS = 2
_NG = 64
_VOCAB = 1001

_NP = 10240
_VP = 1024
_BN = 256
_KB = 64
_EP = 331776
_TBLK = _EP // 16 // _KB
_DRW = _NP // 16
_NR = 8
_RNG = _NP // _NR
_TRASH = 32
_ACCR = _RNG + _TRASH
_RPTR = _RNG // 16
_NEG_SLOPE = 0.2
_INV_SQRT2 = 0.7071067811865476


def _sc_gat(h_tab, als, ald, srcs, dsts):
  mesh = plsc.VectorSubcoreMesh(core_axis_name="c", subcore_axis_name="s")

  @functools.partial(
      pl.kernel,
      out_type=(jax.ShapeDtypeStruct((2 * _NP, _HID), jnp.float32),
                jax.ShapeDtypeStruct((32, _DRW, 16), jnp.float32)),
      mesh=mesh,
      compiler_params=pltpu.CompilerParams(needs_layout_passes=False),
      scratch_types=[
          pltpu.VMEM((_NP,), jnp.float32),
          pltpu.VMEM((_NP,), jnp.float32),
          pltpu.VMEM((_DRW, 16), jnp.float32),
          pltpu.VMEM((_KB,), jnp.int32),
          pltpu.VMEM((_KB,), jnp.int32),
          pltpu.VMEM((_KB,), jnp.int32),
          pltpu.VMEM((_KB,), jnp.int32),
          pltpu.VMEM((_KB,), jnp.float32),
          pltpu.VMEM((_KB, _HID), jnp.float32),
          pltpu.VMEM((64, 16), jnp.float32),
          pltpu.VMEM_SHARED((_ACCR, _HID), jnp.float32),
          pltpu.SemaphoreType.DMA,
      ])
  def k(h_ref, als_ref, ald_ref, src_ref, dst_ref, out_ref, denp_ref,
        als_v, ald_v, den_v, src_v, dst_v, gsrc_v, dloc_v, coef_v, rows_v,
        tmp_v, acc_sh, sem):
    c = lax.axis_index("c")
    s = lax.axis_index("s")
    z16 = jnp.zeros((16,), jnp.float32)
    iota16 = lax.iota(jnp.int32, 16)

    def zero_rows(i, carry):
      for j in range(_HID // 16):
        rows_v[i, pl.ds(j * 16, 16)] = z16
      return carry

    def zero_den(i, carry):
      den_v[i, :] = z16
      return carry
    lax.fori_loop(0, _DRW, zero_den, 0)

    tab_off = pl.multiple_of(c * _NP, 8)
    pltpu.sync_copy(als_ref.at[pl.ds(tab_off, _NP)], als_v)
    pltpu.sync_copy(ald_ref.at[pl.ds(tab_off, _NP)], ald_v)
    plsc.subcore_barrier()

    def edge_block_ptrs(b):
      return pl.multiple_of((s * _TBLK + b) * _KB, _KB)

    def edge_weights(j):
      sj = src_v[pl.ds(j * 16, 16)]
      dj = dst_v[pl.ds(j * 16, 16)]
      a = plsc.load_gather(als_v, [sj]) + plsc.load_gather(ald_v, [dj])
      a = jnp.where(a >= 0.0, a, _NEG_SLOPE * a)
      ex = jnp.exp(a)
      r = lax.shift_right_logical(dj, 4)
      cl = lax.bitwise_and(dj, 15)
      return sj, dj, r, cl, ex

    def p1(b, carry):
      base = edge_block_ptrs(b)
      pltpu.sync_copy(src_ref.at[pl.ds(base, _KB)], src_v)
      pltpu.sync_copy(dst_ref.at[pl.ds(base, _KB)], dst_v)
      for j in range(_KB // 16):
        _, _, r, cl, ex = edge_weights(j)
        plsc.addupdate_scatter(den_v, [r, cl], ex)
      return carry
    lax.fori_loop(0, _TBLK, p1, 0)

    pltpu.sync_copy(den_v, denp_ref.at[c * 16 + s])
    plsc.subcore_barrier()
    lax.fori_loop(0, _DRW, zero_den, 0)

    def dred(ch, carry):
      base = pl.multiple_of(ch * 64, 8)

      def dslot(t, carry2):
        pltpu.sync_copy(denp_ref.at[c * 16 + t, pl.ds(base, 64), :], tmp_v)

        def dacc(i, carry3):
          den_v[base + i, :] = den_v[base + i, :] + tmp_v[i, :]
          return carry3
        lax.fori_loop(0, 64, dacc, 0)
        return carry2
      lax.fori_loop(0, 16, dslot, 0)
      return carry
    lax.fori_loop(0, _DRW // 64, dred, 0)

    def sweep(rng_i, carry0):
      lo = rng_i * _RNG
      lax.fori_loop(0, _KB, zero_rows, 0)
      for zc in range(_RPTR // _KB):
        pltpu.sync_copy(rows_v, acc_sh.at[pl.ds(s * _RPTR + zc * _KB, _KB), :])
      if _RPTR % _KB:
        pltpu.sync_copy(
            rows_v.at[pl.ds(0, _RPTR % _KB), :],
            acc_sh.at[pl.ds(s * _RPTR + (_RPTR // _KB) * _KB, _RPTR % _KB), :])
      plsc.subcore_barrier()

      def p2(b, carry):
        base = edge_block_ptrs(b)
        pltpu.sync_copy(src_ref.at[pl.ds(base, _KB)], src_v)
        pltpu.sync_copy(dst_ref.at[pl.ds(base, _KB)], dst_v)
        for j in range(_KB // 16):
          sj, dj, r, cl, ex = edge_weights(j)
          den = plsc.load_gather(den_v, [r, cl])
          valid = jnp.logical_and(dj >= lo, dj < lo + _RNG)
          coef_v[pl.ds(j * 16, 16)] = jnp.where(
              valid, ex / (den + 1e-16), 0.0)
          park = _RNG + ((iota16 + j * 16) & (_TRASH - 1))
          dloc_v[pl.ds(j * 16, 16)] = jnp.where(valid, dj - lo, park)
          gsrc_v[pl.ds(j * 16, 16)] = sj + c * _NP
        pltpu.async_copy(h_ref.at[gsrc_v], rows_v, sem).wait()

        def scale(i, carry2):
          cv = plsc.load_gather(coef_v, [jnp.full((16,), i, jnp.int32)])
          for j in range(_HID // 16):
            rows_v[i, pl.ds(j * 16, 16)] = rows_v[i, pl.ds(j * 16, 16)] * cv
          return carry2
        lax.fori_loop(0, _KB, scale, 0)

        pltpu.sync_copy(rows_v, acc_sh.at[dloc_v], add=True)
        return carry
      lax.fori_loop(0, _TBLK, p2, 0)
      plsc.subcore_barrier()

      out_row = pl.multiple_of(c * _NP + lo + s * _RPTR, 8)
      pltpu.sync_copy(
          acc_sh.at[pl.ds(s * _RPTR, _RPTR), :],
          out_ref.at[pl.ds(out_row, _RPTR), :])
      plsc.subcore_barrier()
      return carry0
    lax.fori_loop(0, _NR, sweep, 0)

  return k(h_tab, als, ald, srcs, dsts)[0]


def _sc_gat_xla(h_tab, als, ald, srcs, dsts):
  gsrc0 = srcs
  gsrc1 = srcs + _NP
  a0 = als[gsrc0] + ald[dsts]
  a1 = als[gsrc1] + ald[_NP + dsts]
  a = jnp.stack([a0, a1], axis=1)
  a = jnp.where(a >= 0, a, _NEG_SLOPE * a)
  ex = jnp.exp(a)
  dh = jnp.concatenate([dsts, _NP + dsts])
  exf = jnp.concatenate([ex[:, 0], ex[:, 1]])
  den = jax.ops.segment_sum(exf, dh, num_segments=2 * _NP)
  coef = exf / (den[dh] + 1e-16)
  rows = h_tab[jnp.concatenate([gsrc0, gsrc1])]
  out = jax.ops.segment_sum(rows * coef[:, None], dh, num_segments=2 * _NP)
  return out


def _dot(a, b):
  return jnp.dot(a, b, preferred_element_type=jnp.float32,
                 precision=lax.Precision.HIGHEST)


def _logits(h0, h1, av):
  als0 = jnp.sum(h0 * av[0:1, :], axis=1)
  als1 = jnp.sum(h1 * av[1:2, :], axis=1)
  ald0 = jnp.sum(h0 * av[2:3, :], axis=1)
  ald1 = jnp.sum(h1 * av[3:4, :], axis=1)
  return jnp.concatenate(
      [als0[None, :], als1[None, :], ald0[None, :], ald1[None, :]], axis=0)


def _tc_layer1(xp, embp, w1a, w1b, avec):
  nblk = _NP // _BN

  def body(x_ref, emb_ref, wa_ref, wb_ref, av_ref, h_ref, al_ref):
    xb = x_ref[:, :]
    ioh = lax.broadcasted_iota(jnp.int32, (_BN, _VP), 1)
    oh = (xb == ioh).astype(jnp.float32)
    xe = _dot(oh, emb_ref[:, :])
    h0 = _dot(xe, wa_ref[:, :])
    h1 = _dot(xe, wb_ref[:, :])
    h_ref[0] = h0
    h_ref[1] = h1
    al_ref[:, :] = _logits(h0, h1, av_ref[:, :])

  return pl.pallas_call(
      body,
      grid=(nblk,),
      in_specs=[
          pl.BlockSpec((_BN, 1), lambda i: (i, 0)),
          pl.BlockSpec((_VP, _HID), lambda i: (0, 0)),
          pl.BlockSpec((_HID, _HID), lambda i: (0, 0)),
          pl.BlockSpec((_HID, _HID), lambda i: (0, 0)),
          pl.BlockSpec((4, _HID), lambda i: (0, 0)),
      ],
      out_specs=[
          pl.BlockSpec((2, _BN, _HID), lambda i: (0, i, 0)),
          pl.BlockSpec((4, _BN), lambda i: (0, i)),
      ],
      out_shape=[
          jax.ShapeDtypeStruct((2, _NP, _HID), jnp.float32),
          jax.ShapeDtypeStruct((4, _NP), jnp.float32),
      ],
  )(xp, embp, w1a, w1b, avec)


def _tc_mid(agg, b1r, lgr, lbr, w2aa, w2ba, w2ab, w2bb, avec2):
  nblk = _NP // _BN

  def body(a_ref, b1_ref, g_ref, lb_ref, waa_ref, wba_ref, wab_ref, wbb_ref,
           av_ref, h_ref, al_ref):
    a0 = a_ref[0] + b1_ref[0:1, :]
    a1 = a_ref[1] + b1_ref[1:2, :]
    mu = (jnp.sum(a0, 1, keepdims=True) +
          jnp.sum(a1, 1, keepdims=True)) * (1.0 / 256.0)
    ms = (jnp.sum(a0 * a0, 1, keepdims=True) +
          jnp.sum(a1 * a1, 1, keepdims=True)) * (1.0 / 256.0)
    inv = lax.rsqrt(ms - mu * mu + 1e-5)
    x0 = (a0 - mu) * inv * g_ref[0:1, :] + lb_ref[0:1, :]
    x1 = (a1 - mu) * inv * g_ref[1:2, :] + lb_ref[1:2, :]
    g0 = 0.5 * x0 * (1.0 + lax.erf(x0 * _INV_SQRT2))
    g1 = 0.5 * x1 * (1.0 + lax.erf(x1 * _INV_SQRT2))
    h0 = _dot(g0, waa_ref[:, :]) + _dot(g1, wba_ref[:, :])
    h1 = _dot(g0, wab_ref[:, :]) + _dot(g1, wbb_ref[:, :])
    h_ref[0] = h0
    h_ref[1] = h1
    al_ref[:, :] = _logits(h0, h1, av_ref[:, :])

  full = lambda shape: pl.BlockSpec(shape, lambda i: tuple(0 for _ in shape))
  return pl.pallas_call(
      body,
      grid=(nblk,),
      in_specs=[
          pl.BlockSpec((2, _BN, _HID), lambda i: (0, i, 0)),
          full((2, _HID)),
          full((2, _HID)),
          full((2, _HID)),
          full((_HID, _HID)),
          full((_HID, _HID)),
          full((_HID, _HID)),
          full((_HID, _HID)),
          full((4, _HID)),
      ],
      out_specs=[
          pl.BlockSpec((2, _BN, _HID), lambda i: (0, i, 0)),
          pl.BlockSpec((4, _BN), lambda i: (0, i)),
      ],
      out_shape=[
          jax.ShapeDtypeStruct((2, _NP, _HID), jnp.float32),
          jax.ShapeDtypeStruct((4, _NP), jnp.float32),
      ],
  )(agg, b1r, lgr, lbr, w2aa, w2ba, w2ab, w2bb, avec2)


def _tc_final(agg2, b2r, w3a, w3b, b3r, bidxp):
  nblk = _NP // _BN

  def body(a_ref, b2_ref, wa_ref, wb_ref, b3_ref, bi_ref, h_ref, z_ref, zacc):
    i = pl.program_id(0)
    a0 = a_ref[0] + b2_ref[0:1, :]
    a1 = a_ref[1] + b2_ref[1:2, :]
    hh = _dot(a0, wa_ref[:, :]) + _dot(a1, wb_ref[:, :]) + b3_ref[0:1, :]
    h_ref[:, :] = hh
    oh = (bi_ref[:, :] ==
          lax.broadcasted_iota(jnp.int32, (_BN, _NG), 1)).astype(jnp.float32)
    aug = jnp.concatenate(
        [hh, jnp.ones((_BN, 1), jnp.float32),
         jnp.zeros((_BN, _HID - 1), jnp.float32)], axis=1)
    part = lax.dot_general(oh, aug, (((0,), (0,)), ((), ())),
                           preferred_element_type=jnp.float32,
                           precision=lax.Precision.HIGHEST)

    @pl.when(i == 0)
    def _():
      zacc[:, :] = part

    @pl.when(i > 0)
    def _():
      zacc[:, :] = zacc[:, :] + part

    @pl.when(i == nblk - 1)
    def _():
      acc = zacc[:, :]
      cnt = jnp.maximum(acc[:, _HID:_HID + 1], 1.0)
      z_ref[:, :] = acc[:, :_HID] / cnt

  full = lambda shape: pl.BlockSpec(shape, lambda i: tuple(0 for _ in shape))
  return pl.pallas_call(
      body,
      grid=(nblk,),
      in_specs=[
          pl.BlockSpec((2, _BN, _HID), lambda i: (0, i, 0)),
          full((2, _HID)),
          full((_HID, _HID)),
          full((_HID, _HID)),
          full((1, _HID)),
          pl.BlockSpec((_BN, 1), lambda i: (i, 0)),
      ],
      out_specs=[
          pl.BlockSpec((_BN, _HID), lambda i: (i, 0)),
          pl.BlockSpec((_NG, _HID), lambda i: (0, 0)),
      ],
      out_shape=[
          jax.ShapeDtypeStruct((_NP, _HID), jnp.float32),
          jax.ShapeDtypeStruct((_NG, _HID), jnp.float32),
      ],
      scratch_shapes=[pltpu.VMEM((_NG, 2 * _HID), jnp.float32)],
  )(agg2, b2r, w3a, w3b, b3r, bidxp)


def kernel(x, edge_index, batch_idx, emb, W1, a_src1, a_dst1, b1, ln_g, ln_b,
           W2, a_src2, a_dst2, b2, W3, b3):
  xp = jnp.pad(x.astype(jnp.int32), ((0, _NP - _N), (0, 0)))
  embp = jnp.pad(emb, ((0, _VP - _VOCAB), (0, 0)))

  loops = jnp.arange(_N, dtype=jnp.int32)
  npad = _EP - (_E + _N)
  pad_src = jnp.zeros((npad,), jnp.int32)
  pad_dst = _N + (jnp.arange(npad, dtype=jnp.int32) % (_NP - _N))
  srcs = jnp.concatenate([edge_index[0].astype(jnp.int32), loops, pad_src])
  dsts = jnp.concatenate([edge_index[1].astype(jnp.int32), loops, pad_dst])

  avec1 = jnp.concatenate([a_src1.reshape(_HEADS, _HID),
                           a_dst1.reshape(_HEADS, _HID)], axis=0)
  avec2 = jnp.concatenate([a_src2.reshape(_HEADS, _HID),
                           a_dst2.reshape(_HEADS, _HID)], axis=0)

  h_tab1, alv1 = _tc_layer1(xp, embp, W1[:, :_HID], W1[:, _HID:], avec1)
  agg1 = _sc_gat(h_tab1.reshape(2 * _NP, _HID),
                 alv1[0:2].reshape(2 * _NP), alv1[2:4].reshape(2 * _NP),
                 srcs, dsts)

  h_tab2, alv2 = _tc_mid(agg1.reshape(2, _NP, _HID), b1.reshape(2, _HID),
                         ln_g.reshape(2, _HID), ln_b.reshape(2, _HID),
                         W2[:_HID, :_HID], W2[_HID:, :_HID],
                         W2[:_HID, _HID:], W2[_HID:, _HID:], avec2)
  agg2 = _sc_gat(h_tab2.reshape(2 * _NP, _HID),
                 alv2[0:2].reshape(2 * _NP), alv2[2:4].reshape(2 * _NP),
                 srcs, dsts)

  bidxp = jnp.concatenate(
      [batch_idx.astype(jnp.int32),
       jnp.full((_NP - _N,), _NG, jnp.int32)]).reshape(_NP, 1)
  h_out, z = _tc_final(agg2.reshape(2, _NP, _HID), b2.reshape(2, _HID),
                       W3[:_HID, :], W3[_HID:, :], b3.reshape(1, _HID), bidxp)
  return (h_out[:_N], z)


# speedup vs baseline: 4.9341x; 1.0479x over previous
import functools

import jax
import jax.numpy as jnp
from jax import lax
from jax.experimental import pallas as pl
from jax.experimental.pallas import tpu as pltpu
from jax.experimental.pallas import tpu_sc as plsc

_N = 10000
_E = 320000
_HID = 128
_HEADS = 2
_NG = 64
_VOCAB = 1001

_NP = 10240
_VP = 1024
_BN = 256
_KB = 32
_EP = 331776
_TBLK = _EP // 16 // _KB
_DRW = _NP // 16
_NR = 8
_RNG = _NP // _NR
_TRASH = 32
_ACCR = _RNG + _TRASH
_RPTR = _RNG // 16
_NEG_SLOPE = 0.2
_INV_SQRT2 = 0.7071067811865476


def _sc_gat(h_tab, als, ald, srcs, dsts):
  mesh = plsc.VectorSubcoreMesh(core_axis_name="c", subcore_axis_name="s")

  @functools.partial(
      pl.kernel,
      out_type=(jax.ShapeDtypeStruct((2 * _NP, _HID), jnp.float32),
                jax.ShapeDtypeStruct((32, _DRW, 16), jnp.float32)),
      mesh=mesh,
      compiler_params=pltpu.CompilerParams(needs_layout_passes=False),
      scratch_types=[
          pltpu.VMEM((_NP,), jnp.float32),
          pltpu.VMEM((_NP,), jnp.float32),
          pltpu.VMEM((_DRW, 16), jnp.float32),
          pltpu.VMEM((_KB,), jnp.int32),
          pltpu.VMEM((_KB,), jnp.int32),
          pltpu.VMEM((_KB,), jnp.int32),
          pltpu.VMEM((_KB,), jnp.int32),
          pltpu.VMEM((_KB,), jnp.int32),
          pltpu.VMEM((_KB,), jnp.int32),
          pltpu.VMEM((_KB,), jnp.float32),
          pltpu.VMEM((_KB,), jnp.float32),
          pltpu.VMEM((2 * _KB, _HID), jnp.float32),
          pltpu.VMEM((64, 16), jnp.float32),
          pltpu.VMEM_SHARED((_ACCR, _HID), jnp.float32),
          pltpu.SemaphoreType.DMA,
          pltpu.SemaphoreType.DMA,
      ])
  def k(h_ref, als_ref, ald_ref, src_ref, dst_ref, out_ref, denp_ref,
        als_v, ald_v, den_v, src_v, dst_v, gsrc_a, gsrc_b, dloc_a, dloc_b,
        coef_a, coef_b, rows_v, tmp_v, acc_sh, sem_a, sem_b):
    c = lax.axis_index("c")
    s = lax.axis_index("s")
    z16 = jnp.zeros((16,), jnp.float32)
    iota16 = lax.iota(jnp.int32, 16)

    def zero_rows(i, carry):
      for j in range(_HID // 16):
        rows_v[i, pl.ds(j * 16, 16)] = z16
      return carry

    def zero_den(i, carry):
      den_v[i, :] = z16
      return carry
    lax.fori_loop(0, _DRW, zero_den, 0)

    tab_off = pl.multiple_of(c * _NP, 8)
    pltpu.sync_copy(als_ref.at[pl.ds(tab_off, _NP)], als_v)
    pltpu.sync_copy(ald_ref.at[pl.ds(tab_off, _NP)], ald_v)
    plsc.subcore_barrier()

    def edge_block_ptrs(b):
      return pl.multiple_of((s * _TBLK + b) * _KB, _KB)

    def edge_weights(j):
      sj = src_v[pl.ds(j * 16, 16)]
      dj = dst_v[pl.ds(j * 16, 16)]
      a = plsc.load_gather(als_v, [sj]) + plsc.load_gather(ald_v, [dj])
      a = jnp.where(a >= 0.0, a, _NEG_SLOPE * a)
      ex = jnp.exp(a)
      r = lax.shift_right_logical(dj, 4)
      cl = lax.bitwise_and(dj, 15)
      return sj, dj, r, cl, ex

    def p1(b, carry):
      base = edge_block_ptrs(b)
      pltpu.sync_copy(src_ref.at[pl.ds(base, _KB)], src_v)
      pltpu.sync_copy(dst_ref.at[pl.ds(base, _KB)], dst_v)
      for j in range(_KB // 16):
        _, _, r, cl, ex = edge_weights(j)
        plsc.addupdate_scatter(den_v, [r, cl], ex)
      return carry
    lax.fori_loop(0, _TBLK, p1, 0)

    pltpu.sync_copy(den_v, denp_ref.at[c * 16 + s])
    plsc.subcore_barrier()
    lax.fori_loop(0, _DRW, zero_den, 0)

    def dred(ch, carry):
      base = pl.multiple_of(ch * 64, 8)

      def dslot(t, carry2):
        pltpu.sync_copy(denp_ref.at[c * 16 + t, pl.ds(base, 64), :], tmp_v)

        def dacc(i, carry3):
          den_v[base + i, :] = den_v[base + i, :] + tmp_v[i, :]
          return carry3
        lax.fori_loop(0, 64, dacc, 0)
        return carry2
      lax.fori_loop(0, 16, dslot, 0)
      return carry
    lax.fori_loop(0, _DRW // 64, dred, 0)

    def sweep(rng_i, carry0):
      lo = rng_i * _RNG
      lax.fori_loop(0, 2 * _KB, zero_rows, 0)
      rv = 2 * _KB
      for zc in range(_RPTR // rv):
        pltpu.sync_copy(rows_v, acc_sh.at[pl.ds(s * _RPTR + zc * rv, rv), :])
      if _RPTR % rv:
        pltpu.sync_copy(
            rows_v.at[pl.ds(0, _RPTR % rv), :],
            acc_sh.at[pl.ds(s * _RPTR + (_RPTR // rv) * rv, _RPTR % rv), :])
      plsc.subcore_barrier()

      def prep(b, gsrc_x, dloc_x, coef_x, off, sem_x):
        base = edge_block_ptrs(b)
        pltpu.sync_copy(src_ref.at[pl.ds(base, _KB)], src_v)
        pltpu.sync_copy(dst_ref.at[pl.ds(base, _KB)], dst_v)
        for j in range(_KB // 16):
          sj, dj, r, cl, ex = edge_weights(j)
          den = plsc.load_gather(den_v, [r, cl])
          valid = jnp.logical_and(dj >= lo, dj < lo + _RNG)
          coef_x[pl.ds(j * 16, 16)] = jnp.where(
              valid, ex / (den + 1e-16), 0.0)
          park = _RNG + ((iota16 + j * 16) & (_TRASH - 1))
          dloc_x[pl.ds(j * 16, 16)] = jnp.where(valid, dj - lo, park)
          gsrc_x[pl.ds(j * 16, 16)] = sj + c * _NP
        pltpu.async_copy(h_ref.at[gsrc_x], rows_v.at[pl.ds(off, _KB), :], sem_x)

      def finish(gsrc_x, dloc_x, coef_x, off, sem_x):
        pltpu.make_async_copy(
            h_ref.at[gsrc_x], rows_v.at[pl.ds(off, _KB), :], sem_x).wait()

        def scale(i, carry2):
          cv = plsc.load_gather(coef_x, [jnp.full((16,), i, jnp.int32)])
          for j in range(_HID // 16):
            rows_v[off + i, pl.ds(j * 16, 16)] = (
                rows_v[off + i, pl.ds(j * 16, 16)] * cv)
          return carry2
        lax.fori_loop(0, _KB, scale, 0)
        pltpu.sync_copy(rows_v.at[pl.ds(off, _KB), :],
                        acc_sh.at[dloc_x], add=True)

      nb = _TBLK // 2
      prep(0, gsrc_a, dloc_a, coef_a, 0, sem_a)

      def pair(bp, carry):
        prep(bp * 2 + 1, gsrc_b, dloc_b, coef_b, _KB, sem_b)
        finish(gsrc_a, dloc_a, coef_a, 0, sem_a)

        @pl.when(bp < nb - 1)
        def _():
          prep(bp * 2 + 2, gsrc_a, dloc_a, coef_a, 0, sem_a)
        finish(gsrc_b, dloc_b, coef_b, _KB, sem_b)
        return carry
      lax.fori_loop(0, nb, pair, 0)
      plsc.subcore_barrier()

      out_row = pl.multiple_of(c * _NP + lo + s * _RPTR, 8)
      pltpu.sync_copy(
          acc_sh.at[pl.ds(s * _RPTR, _RPTR), :],
          out_ref.at[pl.ds(out_row, _RPTR), :])
      plsc.subcore_barrier()
      return carry0
    lax.fori_loop(0, _NR, sweep, 0)

  return k(h_tab, als, ald, srcs, dsts)[0]


def _dot(a, b):
  return jnp.dot(a, b, preferred_element_type=jnp.float32,
                 precision=lax.Precision.HIGHEST)


def _logits(h0, h1, av):
  als0 = jnp.sum(h0 * av[0:1, :], axis=1)
  als1 = jnp.sum(h1 * av[1:2, :], axis=1)
  ald0 = jnp.sum(h0 * av[2:3, :], axis=1)
  ald1 = jnp.sum(h1 * av[3:4, :], axis=1)
  return jnp.concatenate(
      [als0[None, :], als1[None, :], ald0[None, :], ald1[None, :]], axis=0)


def _tc_layer1(xp, embp, w1a, w1b, avec):
  nblk = _NP // _BN

  def body(x_ref, emb_ref, wa_ref, wb_ref, av_ref, h_ref, al_ref):
    xb = x_ref[:, :]
    ioh = lax.broadcasted_iota(jnp.int32, (_BN, _VP), 1)
    oh = (xb == ioh).astype(jnp.float32)
    xe = _dot(oh, emb_ref[:, :])
    h0 = _dot(xe, wa_ref[:, :])
    h1 = _dot(xe, wb_ref[:, :])
    h_ref[0] = h0
    h_ref[1] = h1
    al_ref[:, :] = _logits(h0, h1, av_ref[:, :])

  return pl.pallas_call(
      body,
      grid=(nblk,),
      in_specs=[
          pl.BlockSpec((_BN, 1), lambda i: (i, 0)),
          pl.BlockSpec((_VP, _HID), lambda i: (0, 0)),
          pl.BlockSpec((_HID, _HID), lambda i: (0, 0)),
          pl.BlockSpec((_HID, _HID), lambda i: (0, 0)),
          pl.BlockSpec((4, _HID), lambda i: (0, 0)),
      ],
      out_specs=[
          pl.BlockSpec((2, _BN, _HID), lambda i: (0, i, 0)),
          pl.BlockSpec((4, _BN), lambda i: (0, i)),
      ],
      out_shape=[
          jax.ShapeDtypeStruct((2, _NP, _HID), jnp.float32),
          jax.ShapeDtypeStruct((4, _NP), jnp.float32),
      ],
  )(xp, embp, w1a, w1b, avec)


def _tc_mid(agg, b1r, lgr, lbr, w2aa, w2ba, w2ab, w2bb, avec2):
  nblk = _NP // _BN

  def body(a_ref, b1_ref, g_ref, lb_ref, waa_ref, wba_ref, wab_ref, wbb_ref,
           av_ref, h_ref, al_ref):
    a0 = a_ref[0] + b1_ref[0:1, :]
    a1 = a_ref[1] + b1_ref[1:2, :]
    mu = (jnp.sum(a0, 1, keepdims=True) +
          jnp.sum(a1, 1, keepdims=True)) * (1.0 / 256.0)
    ms = (jnp.sum(a0 * a0, 1, keepdims=True) +
          jnp.sum(a1 * a1, 1, keepdims=True)) * (1.0 / 256.0)
    inv = lax.rsqrt(ms - mu * mu + 1e-5)
    x0 = (a0 - mu) * inv * g_ref[0:1, :] + lb_ref[0:1, :]
    x1 = (a1 - mu) * inv * g_ref[1:2, :] + lb_ref[1:2, :]
    g0 = 0.5 * x0 * (1.0 + lax.erf(x0 * _INV_SQRT2))
    g1 = 0.5 * x1 * (1.0 + lax.erf(x1 * _INV_SQRT2))
    h0 = _dot(g0, waa_ref[:, :]) + _dot(g1, wba_ref[:, :])
    h1 = _dot(g0, wab_ref[:, :]) + _dot(g1, wbb_ref[:, :])
    h_ref[0] = h0
    h_ref[1] = h1
    al_ref[:, :] = _logits(h0, h1, av_ref[:, :])

  full = lambda shape: pl.BlockSpec(shape, lambda i: tuple(0 for _ in shape))
  return pl.pallas_call(
      body,
      grid=(nblk,),
      in_specs=[
          pl.BlockSpec((2, _BN, _HID), lambda i: (0, i, 0)),
          full((2, _HID)),
          full((2, _HID)),
          full((2, _HID)),
          full((_HID, _HID)),
          full((_HID, _HID)),
          full((_HID, _HID)),
          full((_HID, _HID)),
          full((4, _HID)),
      ],
      out_specs=[
          pl.BlockSpec((2, _BN, _HID), lambda i: (0, i, 0)),
          pl.BlockSpec((4, _BN), lambda i: (0, i)),
      ],
      out_shape=[
          jax.ShapeDtypeStruct((2, _NP, _HID), jnp.float32),
          jax.ShapeDtypeStruct((4, _NP), jnp.float32),
      ],
  )(agg, b1r, lgr, lbr, w2aa, w2ba, w2ab, w2bb, avec2)


def _tc_final(agg2, b2r, w3a, w3b, b3r, bidxp):
  nblk = _NP // _BN

  def body(a_ref, b2_ref, wa_ref, wb_ref, b3_ref, bi_ref, h_ref, z_ref, zacc):
    i = pl.program_id(0)
    a0 = a_ref[0] + b2_ref[0:1, :]
    a1 = a_ref[1] + b2_ref[1:2, :]
    hh = _dot(a0, wa_ref[:, :]) + _dot(a1, wb_ref[:, :]) + b3_ref[0:1, :]
    h_ref[:, :] = hh
    oh = (bi_ref[:, :] ==
          lax.broadcasted_iota(jnp.int32, (_BN, _NG), 1)).astype(jnp.float32)
    aug = jnp.concatenate(
        [hh, jnp.ones((_BN, 1), jnp.float32),
         jnp.zeros((_BN, _HID - 1), jnp.float32)], axis=1)
    part = lax.dot_general(oh, aug, (((0,), (0,)), ((), ())),
                           preferred_element_type=jnp.float32,
                           precision=lax.Precision.HIGHEST)

    @pl.when(i == 0)
    def _():
      zacc[:, :] = part

    @pl.when(i > 0)
    def _():
      zacc[:, :] = zacc[:, :] + part

    @pl.when(i == nblk - 1)
    def _():
      acc = zacc[:, :]
      cnt = jnp.maximum(acc[:, _HID:_HID + 1], 1.0)
      z_ref[:, :] = acc[:, :_HID] / cnt

  full = lambda shape: pl.BlockSpec(shape, lambda i: tuple(0 for _ in shape))
  return pl.pallas_call(
      body,
      grid=(nblk,),
      in_specs=[
          pl.BlockSpec((2, _BN, _HID), lambda i: (0, i, 0)),
          full((2, _HID)),
          full((_HID, _HID)),
          full((_HID, _HID)),
          full((1, _HID)),
          pl.BlockSpec((_BN, 1), lambda i: (i, 0)),
      ],
      out_specs=[
          pl.BlockSpec((_BN, _HID), lambda i: (i, 0)),
          pl.BlockSpec((_NG, _HID), lambda i: (0, 0)),
      ],
      out_shape=[
          jax.ShapeDtypeStruct((_NP, _HID), jnp.float32),
          jax.ShapeDtypeStruct((_NG, _HID), jnp.float32),
      ],
      scratch_shapes=[pltpu.VMEM((_NG, 2 * _HID), jnp.float32)],
  )(agg2, b2r, w3a, w3b, b3r, bidxp)


def kernel(x, edge_index, batch_idx, emb, W1, a_src1, a_dst1, b1, ln_g, ln_b,
           W2, a_src2, a_dst2, b2, W3, b3):
  xp = jnp.pad(x.astype(jnp.int32), ((0, _NP - _N), (0, 0)))
  embp = jnp.pad(emb, ((0, _VP - _VOCAB), (0, 0)))

  loops = jnp.arange(_N, dtype=jnp.int32)
  npad = _EP - (_E + _N)
  pad_src = jnp.zeros((npad,), jnp.int32)
  pad_dst = _N + (jnp.arange(npad, dtype=jnp.int32) % (_NP - _N))
  srcs = jnp.concatenate([edge_index[0].astype(jnp.int32), loops, pad_src])
  dsts = jnp.concatenate([edge_index[1].astype(jnp.int32), loops, pad_dst])

  avec1 = jnp.concatenate([a_src1.reshape(_HEADS, _HID),
                           a_dst1.reshape(_HEADS, _HID)], axis=0)
  avec2 = jnp.concatenate([a_src2.reshape(_HEADS, _HID),
                           a_dst2.reshape(_HEADS, _HID)], axis=0)

  h_tab1, alv1 = _tc_layer1(xp, embp, W1[:, :_HID], W1[:, _HID:], avec1)
  agg1 = _sc_gat(h_tab1.reshape(2 * _NP, _HID),
                 alv1[0:2].reshape(2 * _NP), alv1[2:4].reshape(2 * _NP),
                 srcs, dsts)

  h_tab2, alv2 = _tc_mid(agg1.reshape(2, _NP, _HID), b1.reshape(2, _HID),
                         ln_g.reshape(2, _HID), ln_b.reshape(2, _HID),
                         W2[:_HID, :_HID], W2[_HID:, :_HID],
                         W2[:_HID, _HID:], W2[_HID:, _HID:], avec2)
  agg2 = _sc_gat(h_tab2.reshape(2 * _NP, _HID),
                 alv2[0:2].reshape(2 * _NP), alv2[2:4].reshape(2 * _NP),
                 srcs, dsts)

  bidxp = jnp.concatenate(
      [batch_idx.astype(jnp.int32),
       jnp.full((_NP - _N,), _NG, jnp.int32)]).reshape(_NP, 1)
  h_out, z = _tc_final(agg2.reshape(2, _NP, _HID), b2.reshape(2, _HID),
                       W3[:_HID, :], W3[_HID:, :], b3.reshape(1, _HID), bidxp)
  return (h_out[:_N], z)


# speedup vs baseline: 5.0356x; 1.0206x over previous
import functools

import jax
import jax.numpy as jnp
from jax import lax
from jax.experimental import pallas as pl
from jax.experimental.pallas import tpu as pltpu
from jax.experimental.pallas import tpu_sc as plsc

_N = 10000
_E = 320000
_HID = 128
_HEADS = 2
_NG = 64
_VOCAB = 1001

_NP = 10240
_VP = 1024
_BN = 256
_KB = 32
_EP = 331776
_TBLK = _EP // 16 // _KB
_DRW = _NP // 16
_NR = 8
_RNG = _NP // _NR
_TRASH = 32
_ACCR = _RNG + _TRASH
_RPTR = _RNG // 16
_NEG_SLOPE = 0.2
_INV_SQRT2 = 0.7071067811865476


def _sc_gat(h_tab, als, ald, srcs, dsts):
  mesh = plsc.VectorSubcoreMesh(core_axis_name="c", subcore_axis_name="s")

  @functools.partial(
      pl.kernel,
      out_type=(jax.ShapeDtypeStruct((2 * _NP, _HID), jnp.float32),
                jax.ShapeDtypeStruct((32, _DRW, 16), jnp.float32)),
      mesh=mesh,
      compiler_params=pltpu.CompilerParams(needs_layout_passes=False),
      scratch_types=[
          pltpu.VMEM((_NP,), jnp.float32),
          pltpu.VMEM((_NP,), jnp.float32),
          pltpu.VMEM((_DRW, 16), jnp.float32),
          pltpu.VMEM((_KB,), jnp.int32),
          pltpu.VMEM((_KB,), jnp.int32),
          pltpu.VMEM((_KB,), jnp.int32),
          pltpu.VMEM((_KB,), jnp.int32),
          pltpu.VMEM((_KB,), jnp.int32),
          pltpu.VMEM((_KB,), jnp.int32),
          pltpu.VMEM((_KB,), jnp.float32),
          pltpu.VMEM((_KB,), jnp.float32),
          pltpu.VMEM((2 * _KB, _HID), jnp.float32),
          pltpu.VMEM((64, 16), jnp.float32),
          pltpu.VMEM_SHARED((_ACCR, _HID), jnp.float32),
          pltpu.SemaphoreType.DMA,
          pltpu.SemaphoreType.DMA,
      ])
  def k(h_ref, als_ref, ald_ref, src_ref, dst_ref, out_ref, denp_ref,
        als_v, ald_v, den_v, src_v, dst_v, gsrc_a, gsrc_b, dloc_a, dloc_b,
        coef_a, coef_b, rows_v, tmp_v, acc_sh, sem_a, sem_b):
    c = lax.axis_index("c")
    s = lax.axis_index("s")
    z16 = jnp.zeros((16,), jnp.float32)
    iota16 = lax.iota(jnp.int32, 16)

    def zero_rows(i, carry):
      for j in range(_HID // 16):
        rows_v[i, pl.ds(j * 16, 16)] = z16
      return carry

    def zero_den(i, carry):
      den_v[i, :] = z16
      return carry
    lax.fori_loop(0, _DRW, zero_den, 0)

    tab_off = pl.multiple_of(c * _NP, 8)
    pltpu.sync_copy(als_ref.at[pl.ds(tab_off, _NP)], als_v)
    pltpu.sync_copy(ald_ref.at[pl.ds(tab_off, _NP)], ald_v)
    plsc.subcore_barrier()

    def edge_block_ptrs(b):
      return pl.multiple_of((s * _TBLK + b) * _KB, _KB)

    def edge_weights(j):
      sj = src_v[pl.ds(j * 16, 16)]
      dj = dst_v[pl.ds(j * 16, 16)]
      a = plsc.load_gather(als_v, [sj]) + plsc.load_gather(ald_v, [dj])
      a = jnp.where(a >= 0.0, a, _NEG_SLOPE * a)
      ex = jnp.exp(a)
      r = lax.shift_right_logical(dj, 4)
      cl = lax.bitwise_and(dj, 15)
      return sj, dj, r, cl, ex

    def p1(b, carry):
      base = edge_block_ptrs(b)
      pltpu.sync_copy(src_ref.at[pl.ds(base, _KB)], src_v)
      pltpu.sync_copy(dst_ref.at[pl.ds(base, _KB)], dst_v)
      for j in range(_KB // 16):
        _, _, r, cl, ex = edge_weights(j)
        plsc.addupdate_scatter(den_v, [r, cl], ex)
      return carry
    lax.fori_loop(0, _TBLK, p1, 0)

    pltpu.sync_copy(den_v, denp_ref.at[c * 16 + s])
    plsc.subcore_barrier()
    lax.fori_loop(0, _DRW, zero_den, 0)

    def dred(ch, carry):
      base = pl.multiple_of(ch * 64, 8)

      def dslot(t, carry2):
        pltpu.sync_copy(denp_ref.at[c * 16 + t, pl.ds(base, 64), :], tmp_v)

        def dacc(i, carry3):
          den_v[base + i, :] = den_v[base + i, :] + tmp_v[i, :]
          return carry3
        lax.fori_loop(0, 64, dacc, 0)
        return carry2
      lax.fori_loop(0, 16, dslot, 0)
      return carry
    lax.fori_loop(0, _DRW // 64, dred, 0)

    def sweep(rng_i, carry0):
      lo = rng_i * _RNG
      lax.fori_loop(0, 2 * _KB, zero_rows, 0)
      rv = 2 * _KB
      for zc in range(_RPTR // rv):
        pltpu.sync_copy(rows_v, acc_sh.at[pl.ds(s * _RPTR + zc * rv, rv), :])
      if _RPTR % rv:
        pltpu.sync_copy(
            rows_v.at[pl.ds(0, _RPTR % rv), :],
            acc_sh.at[pl.ds(s * _RPTR + (_RPTR // rv) * rv, _RPTR % rv), :])
      plsc.subcore_barrier()

      def prep(b, gsrc_x, dloc_x, coef_x, off, sem_x):
        base = edge_block_ptrs(b)
        pltpu.sync_copy(src_ref.at[pl.ds(base, _KB)], src_v)
        pltpu.sync_copy(dst_ref.at[pl.ds(base, _KB)], dst_v)
        for j in range(_KB // 16):
          sj, dj, r, cl, ex = edge_weights(j)
          den = plsc.load_gather(den_v, [r, cl])
          valid = jnp.logical_and(dj >= lo, dj < lo + _RNG)
          coef_x[pl.ds(j * 16, 16)] = jnp.where(
              valid, ex / (den + 1e-16), 0.0)
          park = _RNG + ((iota16 + j * 16) & (_TRASH - 1))
          dloc_x[pl.ds(j * 16, 16)] = jnp.where(valid, dj - lo, park)
          gsrc_x[pl.ds(j * 16, 16)] = sj + c * _NP
        pltpu.async_copy(h_ref.at[gsrc_x], rows_v.at[pl.ds(off, _KB), :], sem_x)

      def finish(gsrc_x, dloc_x, coef_x, off, sem_x):
        pltpu.make_async_copy(
            h_ref.at[gsrc_x], rows_v.at[pl.ds(off, _KB), :], sem_x).wait()

        def scale(i4, carry2):
          i0 = i4 * 4
          for u in range(4):
            cv = plsc.load_gather(coef_x, [jnp.full((16,), i0 + u, jnp.int32)])
            for j in range(_HID // 16):
              rows_v[off + i0 + u, pl.ds(j * 16, 16)] = (
                  rows_v[off + i0 + u, pl.ds(j * 16, 16)] * cv)
          return carry2
        lax.fori_loop(0, _KB // 4, scale, 0)
        pltpu.sync_copy(rows_v.at[pl.ds(off, _KB), :],
                        acc_sh.at[dloc_x], add=True)

      nb = _TBLK // 2
      prep(0, gsrc_a, dloc_a, coef_a, 0, sem_a)

      def pair(bp, carry):
        prep(bp * 2 + 1, gsrc_b, dloc_b, coef_b, _KB, sem_b)
        finish(gsrc_a, dloc_a, coef_a, 0, sem_a)

        @pl.when(bp < nb - 1)
        def _():
          prep(bp * 2 + 2, gsrc_a, dloc_a, coef_a, 0, sem_a)
        finish(gsrc_b, dloc_b, coef_b, _KB, sem_b)
        return carry
      lax.fori_loop(0, nb, pair, 0)
      plsc.subcore_barrier()

      out_row = pl.multiple_of(c * _NP + lo + s * _RPTR, 8)
      pltpu.sync_copy(
          acc_sh.at[pl.ds(s * _RPTR, _RPTR), :],
          out_ref.at[pl.ds(out_row, _RPTR), :])
      plsc.subcore_barrier()
      return carry0
    lax.fori_loop(0, _NR, sweep, 0)

  return k(h_tab, als, ald, srcs, dsts)[0]


def _dot(a, b):
  return jnp.dot(a, b, preferred_element_type=jnp.float32,
                 precision=lax.Precision.HIGHEST)


def _logits(h0, h1, av):
  als0 = jnp.sum(h0 * av[0:1, :], axis=1)
  als1 = jnp.sum(h1 * av[1:2, :], axis=1)
  ald0 = jnp.sum(h0 * av[2:3, :], axis=1)
  ald1 = jnp.sum(h1 * av[3:4, :], axis=1)
  return jnp.concatenate(
      [als0[None, :], als1[None, :], ald0[None, :], ald1[None, :]], axis=0)


def _tc_layer1(xp, embp, w1a, w1b, avec):
  nblk = _NP // _BN

  def body(x_ref, emb_ref, wa_ref, wb_ref, av_ref, h_ref, al_ref):
    xb = x_ref[:, :]
    ioh = lax.broadcasted_iota(jnp.int32, (_BN, _VP), 1)
    oh = (xb == ioh).astype(jnp.float32)
    xe = _dot(oh, emb_ref[:, :])
    h0 = _dot(xe, wa_ref[:, :])
    h1 = _dot(xe, wb_ref[:, :])
    h_ref[0] = h0
    h_ref[1] = h1
    al_ref[:, :] = _logits(h0, h1, av_ref[:, :])

  return pl.pallas_call(
      body,
      grid=(nblk,),
      in_specs=[
          pl.BlockSpec((_BN, 1), lambda i: (i, 0)),
          pl.BlockSpec((_VP, _HID), lambda i: (0, 0)),
          pl.BlockSpec((_HID, _HID), lambda i: (0, 0)),
          pl.BlockSpec((_HID, _HID), lambda i: (0, 0)),
          pl.BlockSpec((4, _HID), lambda i: (0, 0)),
      ],
      out_specs=[
          pl.BlockSpec((2, _BN, _HID), lambda i: (0, i, 0)),
          pl.BlockSpec((4, _BN), lambda i: (0, i)),
      ],
      out_shape=[
          jax.ShapeDtypeStruct((2, _NP, _HID), jnp.float32),
          jax.ShapeDtypeStruct((4, _NP), jnp.float32),
      ],
  )(xp, embp, w1a, w1b, avec)


def _tc_mid(agg, b1r, lgr, lbr, w2aa, w2ba, w2ab, w2bb, avec2):
  nblk = _NP // _BN

  def body(a_ref, b1_ref, g_ref, lb_ref, waa_ref, wba_ref, wab_ref, wbb_ref,
           av_ref, h_ref, al_ref):
    a0 = a_ref[0] + b1_ref[0:1, :]
    a1 = a_ref[1] + b1_ref[1:2, :]
    mu = (jnp.sum(a0, 1, keepdims=True) +
          jnp.sum(a1, 1, keepdims=True)) * (1.0 / 256.0)
    ms = (jnp.sum(a0 * a0, 1, keepdims=True) +
          jnp.sum(a1 * a1, 1, keepdims=True)) * (1.0 / 256.0)
    inv = lax.rsqrt(ms - mu * mu + 1e-5)
    x0 = (a0 - mu) * inv * g_ref[0:1, :] + lb_ref[0:1, :]
    x1 = (a1 - mu) * inv * g_ref[1:2, :] + lb_ref[1:2, :]
    g0 = 0.5 * x0 * (1.0 + lax.erf(x0 * _INV_SQRT2))
    g1 = 0.5 * x1 * (1.0 + lax.erf(x1 * _INV_SQRT2))
    h0 = _dot(g0, waa_ref[:, :]) + _dot(g1, wba_ref[:, :])
    h1 = _dot(g0, wab_ref[:, :]) + _dot(g1, wbb_ref[:, :])
    h_ref[0] = h0
    h_ref[1] = h1
    al_ref[:, :] = _logits(h0, h1, av_ref[:, :])

  full = lambda shape: pl.BlockSpec(shape, lambda i: tuple(0 for _ in shape))
  return pl.pallas_call(
      body,
      grid=(nblk,),
      in_specs=[
          pl.BlockSpec((2, _BN, _HID), lambda i: (0, i, 0)),
          full((2, _HID)),
          full((2, _HID)),
          full((2, _HID)),
          full((_HID, _HID)),
          full((_HID, _HID)),
          full((_HID, _HID)),
          full((_HID, _HID)),
          full((4, _HID)),
      ],
      out_specs=[
          pl.BlockSpec((2, _BN, _HID), lambda i: (0, i, 0)),
          pl.BlockSpec((4, _BN), lambda i: (0, i)),
      ],
      out_shape=[
          jax.ShapeDtypeStruct((2, _NP, _HID), jnp.float32),
          jax.ShapeDtypeStruct((4, _NP), jnp.float32),
      ],
  )(agg, b1r, lgr, lbr, w2aa, w2ba, w2ab, w2bb, avec2)


def _tc_final(agg2, b2r, w3a, w3b, b3r, bidxp):
  nblk = _NP // _BN

  def body(a_ref, b2_ref, wa_ref, wb_ref, b3_ref, bi_ref, h_ref, z_ref, zacc):
    i = pl.program_id(0)
    a0 = a_ref[0] + b2_ref[0:1, :]
    a1 = a_ref[1] + b2_ref[1:2, :]
    hh = _dot(a0, wa_ref[:, :]) + _dot(a1, wb_ref[:, :]) + b3_ref[0:1, :]
    h_ref[:, :] = hh
    oh = (bi_ref[:, :] ==
          lax.broadcasted_iota(jnp.int32, (_BN, _NG), 1)).astype(jnp.float32)
    aug = jnp.concatenate(
        [hh, jnp.ones((_BN, 1), jnp.float32),
         jnp.zeros((_BN, _HID - 1), jnp.float32)], axis=1)
    part = lax.dot_general(oh, aug, (((0,), (0,)), ((), ())),
                           preferred_element_type=jnp.float32,
                           precision=lax.Precision.HIGHEST)

    @pl.when(i == 0)
    def _():
      zacc[:, :] = part

    @pl.when(i > 0)
    def _():
      zacc[:, :] = zacc[:, :] + part

    @pl.when(i == nblk - 1)
    def _():
      acc = zacc[:, :]
      cnt = jnp.maximum(acc[:, _HID:_HID + 1], 1.0)
      z_ref[:, :] = acc[:, :_HID] / cnt

  full = lambda shape: pl.BlockSpec(shape, lambda i: tuple(0 for _ in shape))
  return pl.pallas_call(
      body,
      grid=(nblk,),
      in_specs=[
          pl.BlockSpec((2, _BN, _HID), lambda i: (0, i, 0)),
          full((2, _HID)),
          full((_HID, _HID)),
          full((_HID, _HID)),
          full((1, _HID)),
          pl.BlockSpec((_BN, 1), lambda i: (i, 0)),
      ],
      out_specs=[
          pl.BlockSpec((_BN, _HID), lambda i: (i, 0)),
          pl.BlockSpec((_NG, _HID), lambda i: (0, 0)),
      ],
      out_shape=[
          jax.ShapeDtypeStruct((_NP, _HID), jnp.float32),
          jax.ShapeDtypeStruct((_NG, _HID), jnp.float32),
      ],
      scratch_shapes=[pltpu.VMEM((_NG, 2 * _HID), jnp.float32)],
  )(agg2, b2r, w3a, w3b, b3r, bidxp)


def kernel(x, edge_index, batch_idx, emb, W1, a_src1, a_dst1, b1, ln_g, ln_b,
           W2, a_src2, a_dst2, b2, W3, b3):
  xp = jnp.pad(x.astype(jnp.int32), ((0, _NP - _N), (0, 0)))
  embp = jnp.pad(emb, ((0, _VP - _VOCAB), (0, 0)))

  loops = jnp.arange(_N, dtype=jnp.int32)
  npad = _EP - (_E + _N)
  pad_src = jnp.zeros((npad,), jnp.int32)
  pad_dst = _N + (jnp.arange(npad, dtype=jnp.int32) % (_NP - _N))
  srcs = jnp.concatenate([edge_index[0].astype(jnp.int32), loops, pad_src])
  dsts = jnp.concatenate([edge_index[1].astype(jnp.int32), loops, pad_dst])

  avec1 = jnp.concatenate([a_src1.reshape(_HEADS, _HID),
                           a_dst1.reshape(_HEADS, _HID)], axis=0)
  avec2 = jnp.concatenate([a_src2.reshape(_HEADS, _HID),
                           a_dst2.reshape(_HEADS, _HID)], axis=0)

  h_tab1, alv1 = _tc_layer1(xp, embp, W1[:, :_HID], W1[:, _HID:], avec1)
  agg1 = _sc_gat(h_tab1.reshape(2 * _NP, _HID),
                 alv1[0:2].reshape(2 * _NP), alv1[2:4].reshape(2 * _NP),
                 srcs, dsts)

  h_tab2, alv2 = _tc_mid(agg1.reshape(2, _NP, _HID), b1.reshape(2, _HID),
                         ln_g.reshape(2, _HID), ln_b.reshape(2, _HID),
                         W2[:_HID, :_HID], W2[_HID:, :_HID],
                         W2[:_HID, _HID:], W2[_HID:, _HID:], avec2)
  agg2 = _sc_gat(h_tab2.reshape(2 * _NP, _HID),
                 alv2[0:2].reshape(2 * _NP), alv2[2:4].reshape(2 * _NP),
                 srcs, dsts)

  bidxp = jnp.concatenate(
      [batch_idx.astype(jnp.int32),
       jnp.full((_NP - _N,), _NG, jnp.int32)]).reshape(_NP, 1)
  h_out, z = _tc_final(agg2.reshape(2, _NP, _HID), b2.reshape(2, _HID),
                       W3[:_HID, :], W3[_HID:, :], b3.reshape(1, _HID), bidxp)
  return (h_out[:_N], z)
